# baseline (speedup 1.0000x reference)
import functools

import jax
import jax.numpy as jnp
from jax import lax
from jax.experimental import pallas as pl
from jax.experimental.pallas import tpu as pltpu
from jax.experimental.pallas import tpu_sc as plsc

Z, H, W = 14, 32, 64
C = 70
NC, NS, LANES = 2, 16, 16
HSC = H // 2
WH = W // 2
CSTARTS = (0, 16, 32, 48, C - LANES)


def _fold_sc_body(x_hbm, tc_hbm, out_hbm, buf, sem_in, sem_out, sem_cp):
    cid = lax.axis_index("c")
    sid = lax.axis_index("s")
    wid = sid * NC + cid
    h = wid // 2
    w0 = (wid % 2) * WH

    cp = pltpu.async_copy(
        tc_hbm.at[0, :, h, pl.ds(w0, WH), :],
        out_hbm.at[0, :, h + HSC, pl.ds(w0, WH), :],
        sem_cp,
    )

    pltpu.async_copy(
        x_hbm.at[0, :, h, pl.ds(w0, WH), :], buf, sem_in
    ).wait()

    ch = jnp.minimum(h + 1, H - 2) - jnp.maximum(h - 1, 1) + 1
    chv = jnp.full((LANES,), ch).astype(jnp.float32)

    def z_iter(z, carry):
        cz = jnp.minimum(z + 1, Z - 2) - jnp.maximum(z - 1, 1) + 1
        cnt = chv * (cz * 3).astype(jnp.float32)
        rcp = 1.0 / cnt
        for w in range(WH):
            for c0 in CSTARTS:
                sl = pl.ds(c0, LANES)
                acc = buf[z, w, sl] * cnt
                buf[z, w, sl] = acc * rcp
        return carry

    lax.fori_loop(0, Z, z_iter, 0)

    pltpu.async_copy(
        buf, out_hbm.at[0, :, h, pl.ds(w0, WH), :], sem_out
    ).wait()
    cp.wait()


@functools.partial(
    pl.kernel,
    mesh=plsc.VectorSubcoreMesh(core_axis_name="c", subcore_axis_name="s"),
    out_type=jax.ShapeDtypeStruct((1, Z, H, W, C), jnp.float32),
    scratch_types=[
        pltpu.VMEM((Z, WH, C), jnp.float32),
        pltpu.SemaphoreType.DMA,
        pltpu.SemaphoreType.DMA,
        pltpu.SemaphoreType.DMA,
    ],
)
def _fold_sc(x_hbm, tc_hbm, out_hbm, buf, sem_in, sem_out, sem_cp):
    _fold_sc_body(x_hbm, tc_hbm, out_hbm, buf, sem_in, sem_out, sem_cp)


def _fold_tc_kernel(x_ref, o_ref):
    h = pl.program_id(0) + HSC
    ch = jnp.minimum(h + 1, H - 2) - jnp.maximum(h - 1, 1) + 1
    zi = lax.broadcasted_iota(jnp.int32, (1, Z, 1, 1, 1), 1)
    cz = jnp.minimum(zi + 1, Z - 2) - jnp.maximum(zi - 1, 1) + 1
    cnt = (cz * ch * 3).astype(jnp.float32)
    acc = x_ref[...] * cnt
    o_ref[...] = acc / cnt


def _fold_tc(xt):
    return pl.pallas_call(
        _fold_tc_kernel,
        grid=(H - HSC,),
        in_specs=[
            pl.BlockSpec((1, Z, 1, W, C), lambda i: (0, 0, i + HSC, 0, 0)),
        ],
        out_specs=pl.BlockSpec((1, Z, 1, W, C), lambda i: (0, 0, i, 0, 0)),
        out_shape=jax.ShapeDtypeStruct((1, Z, H - HSC, W, C), jnp.float32),
    )(xt)


def kernel(x):
    xt = jnp.transpose(x, (0, 2, 3, 4, 1))
    tc_half = _fold_tc(xt)
    yt = _fold_sc(xt, tc_half)
    return jnp.transpose(yt, (0, 4, 1, 2, 3))


# speedup vs baseline: 7.8417x; 7.8417x over previous
import functools

import jax
import jax.numpy as jnp
from jax import lax
from jax.experimental import pallas as pl
from jax.experimental.pallas import tpu as pltpu
from jax.experimental.pallas import tpu_sc as plsc

Z, H, W = 14, 32, 64
C = 70
NC, NS, LANES = 2, 16, 16
CSTARTS = (0, 16, 32, 48, C - LANES)
ZH = Z // 2


def _fold_body(x_hbm, out_hbm, buf_a, buf_b, sem_a, sem_b):
    cid = lax.axis_index("c")
    sid = lax.axis_index("s")
    h = sid * NC + cid

    in_a = pltpu.async_copy(x_hbm.at[0, pl.ds(0, ZH), h, :, :], buf_a, sem_a)
    in_b = pltpu.async_copy(x_hbm.at[0, pl.ds(ZH, ZH), h, :, :], buf_b, sem_b)

    ch = jnp.minimum(h + 1, H - 2) - jnp.maximum(h - 1, 1) + 1

    def make_z_iter(buf, zoff):
        def z_iter(zi, carry):
            z = zi + zoff
            cz = jnp.minimum(z + 1, Z - 2) - jnp.maximum(z - 1, 1) + 1
            cnt = jnp.full((LANES,), (cz * ch * 3).astype(jnp.float32))
            rcp = 1.0 / cnt
            for w in range(W):
                for c0 in CSTARTS:
                    sl = pl.ds(c0, LANES)
                    acc = buf[zi, w, sl] * cnt
                    buf[zi, w, sl] = acc * rcp
            return carry
        return z_iter

    in_a.wait()
    lax.fori_loop(0, ZH, make_z_iter(buf_a, 0), 0)
    out_a = pltpu.async_copy(buf_a, out_hbm.at[0, pl.ds(0, ZH), h, :, :], sem_a)

    in_b.wait()
    lax.fori_loop(0, ZH, make_z_iter(buf_b, ZH), 0)
    out_b = pltpu.async_copy(buf_b, out_hbm.at[0, pl.ds(ZH, ZH), h, :, :], sem_b)

    out_a.wait()
    out_b.wait()


@functools.partial(
    pl.kernel,
    mesh=plsc.VectorSubcoreMesh(core_axis_name="c", subcore_axis_name="s"),
    out_type=jax.ShapeDtypeStruct((1, Z, H, W, C), jnp.float32),
    scratch_types=[
        pltpu.VMEM((ZH, W, C), jnp.float32),
        pltpu.VMEM((ZH, W, C), jnp.float32),
        pltpu.SemaphoreType.DMA,
        pltpu.SemaphoreType.DMA,
    ],
)
def _fold_sc(x_hbm, out_hbm, buf_a, buf_b, sem_a, sem_b):
    _fold_body(x_hbm, out_hbm, buf_a, buf_b, sem_a, sem_b)


def kernel(x):
    xt = jnp.transpose(x, (0, 2, 3, 4, 1))
    yt = _fold_sc(xt)
    return jnp.transpose(yt, (0, 4, 1, 2, 3))
